# baseline (speedup 1.0000x reference)
import functools

import jax
import jax.numpy as jnp
from jax import lax
from jax.experimental import pallas as pl
from jax.experimental.pallas import tpu as pltpu
from jax.experimental.pallas import tpu_sc as plsc

NC = 2
NS = 16
NW = NC * NS
L = 16

BATCH = 16384
D = 64
DC = 2 * D
B_PER_W = BATCH // NW
CHUNK = 128
NCHUNK = B_PER_W // CHUNK
NGROUP = CHUNK // L


def _sc_body(h_hbm, r_hbm, t_hbm, ent_hbm, rel_hbm,
             out_hbm, idx_h, idx_r, idx_t,
             hc_b, tc_b, rc_b, out_v, sem):
    wid = lax.axis_index("s") * NC + lax.axis_index("c")
    base = wid * B_PER_W

    rows0 = lax.iota(jnp.int32, L)

    for c in range(NCHUNK):
        off = base + c * CHUNK
        pltpu.sync_copy(h_hbm.at[pl.ds(off, CHUNK)], idx_h)
        pltpu.sync_copy(r_hbm.at[pl.ds(off, CHUNK)], idx_r)
        pltpu.sync_copy(t_hbm.at[pl.ds(off, CHUNK)], idx_t)

        copies = [
            pltpu.async_copy(ent_hbm.at[idx_h], hc_b, sem),
            pltpu.async_copy(ent_hbm.at[idx_t], tc_b, sem),
            pltpu.async_copy(rel_hbm.at[idx_r], rc_b, sem),
        ]
        for cp in copies:
            cp.wait()

        def group_compute(g, carry):
            def row_step(j, out_vec):
                i = g * L + j
                acc = jnp.zeros((L,), jnp.float32)
                for s in range(D // L):
                    sl = pl.ds(s * L, L)
                    sh = pl.ds(D + s * L, L)
                    hr = hc_b[i, sl]
                    hi = hc_b[i, sh]
                    tr = tc_b[i, sl]
                    ti = tc_b[i, sh]
                    rr = rc_b[i, sl]
                    ri = rc_b[i, sh]
                    a = hr * rr - hi * ri
                    b = hi * rr + hr * ri
                    acc = acc + a * tr + b * ti
                return jnp.where(rows0 == j, jnp.sum(acc), out_vec)

            out_vec = lax.fori_loop(0, L, row_step,
                                    jnp.zeros((L,), jnp.float32))
            out_v[pl.ds(c * CHUNK + g * L, L)] = out_vec
            return carry

        lax.fori_loop(0, NGROUP, group_compute, 0)

    pltpu.sync_copy(out_v, out_hbm.at[pl.ds(base, B_PER_W)])


@jax.jit
def _complex_score(h, r, t, entity_re, entity_im, relation_re, relation_im):
    ent = jnp.concatenate([entity_re, entity_im], axis=1)
    rel = jnp.concatenate([relation_re, relation_im], axis=1)
    mesh = plsc.VectorSubcoreMesh(core_axis_name="c", subcore_axis_name="s")
    run = functools.partial(
        pl.kernel,
        out_type=jax.ShapeDtypeStruct((BATCH,), jnp.float32),
        mesh=mesh,
        compiler_params=pltpu.CompilerParams(needs_layout_passes=False),
        scratch_types=[
            pltpu.VMEM((CHUNK,), jnp.int32),
            pltpu.VMEM((CHUNK,), jnp.int32),
            pltpu.VMEM((CHUNK,), jnp.int32),
            pltpu.VMEM((CHUNK, DC), jnp.float32),
            pltpu.VMEM((CHUNK, DC), jnp.float32),
            pltpu.VMEM((CHUNK, DC), jnp.float32),
            pltpu.VMEM((B_PER_W,), jnp.float32),
            pltpu.SemaphoreType.DMA,
        ],
    )(_sc_body)
    return run(h, r, t, ent, rel)


def kernel(h, r, t, entity_re, entity_im, relation_re, relation_im):
    return _complex_score(h.astype(jnp.int32), r.astype(jnp.int32),
                          t.astype(jnp.int32), entity_re, entity_im,
                          relation_re, relation_im)
